# baseline (speedup 1.0000x reference)
import functools

import jax
import jax.numpy as jnp
from jax import lax
from jax.experimental import pallas as pl
from jax.experimental.pallas import tpu as pltpu
from jax.experimental.pallas import tpu_sc as plsc

B, L, D = 1024, 200, 64
NUM_LABELS = 100000

_NC, _NS = 2, 16
_NW = _NC * _NS
_RPW = B // _NW
_LA, _LB = 96, 104

def _bag_body(x_hbm, xw_hbm, table_hbm, out_hbm, idx_all, w_all, rows_a,
              rows_b, out_v, sem_a, sem_b):
    wid = lax.axis_index("s") * _NC + lax.axis_index("c")
    base = wid * _RPW * L
    pltpu.sync_copy(x_hbm.at[pl.ds(base, _RPW * L)], idx_all)
    pltpu.sync_copy(xw_hbm.at[pl.ds(base, _RPW * L)], w_all)

    def copies(r, buf, sem):
        off = r * L
        return (
            pltpu.make_async_copy(
                table_hbm.at[idx_all.at[pl.ds(off, _LA)]], rows_a.at[buf], sem),
            pltpu.make_async_copy(
                table_hbm.at[idx_all.at[pl.ds(off + _LA, _LB)]], rows_b.at[buf], sem),
        )

    def issue(r, buf, sem):
        for c in copies(r, buf, sem):
            c.start()

    def accum_range(ref, buf, w_off, l_off, n16, accs):
        def chunk(i, accs):
            w16 = w_all[pl.ds(w_off + i * 16, 16)]
            for u in range(16):
                l = l_off + i * 16 + u
                w = w16[u]
                accs = tuple(
                    a + w * ref[buf, l, pl.ds(16 * k, 16)]
                    for k, a in enumerate(accs))
            return accs
        return lax.fori_loop(0, n16, chunk, accs)

    def row_step(r, buf, sem):
        for c in copies(r, buf, sem):
            c.wait()
        z = jnp.zeros((16,), jnp.float32)
        accs = accum_range(rows_a, buf, r * L, 0, _LA // 16, (z, z, z, z))
        accs = accum_range(rows_b, buf, r * L + _LA, 0, 6, accs)
        w16 = w_all[pl.ds(r * L + L - 16, 16)]
        for u in range(8, 16):
            w = w16[u]
            accs = tuple(
                a + w * rows_b[buf, 88 + u, pl.ds(16 * k, 16)]
                for k, a in enumerate(accs))

        @pl.when(r + 2 < _RPW)
        def _():
            issue(r + 2, buf, sem)
        for k in range(4):
            out_v[pl.ds(r * D + 16 * k, 16)] = accs[k]

    issue(0, 0, sem_a)
    issue(1, 1, sem_b)

    def pair(p, carry):
        row_step(2 * p, 0, sem_a)
        row_step(2 * p + 1, 1, sem_b)
        return carry

    lax.fori_loop(0, _RPW // 2, pair, 0)
    pltpu.sync_copy(out_v, out_hbm.at[pl.ds(wid * _RPW * D, _RPW * D)])


_bag = functools.partial(
    pl.kernel,
    mesh=plsc.VectorSubcoreMesh(core_axis_name="c", subcore_axis_name="s"),
    compiler_params=pltpu.CompilerParams(use_tc_tiling_on_sc=False),
    out_type=jax.ShapeDtypeStruct((B * D,), jnp.float32),
    scratch_types=[
        pltpu.VMEM((_RPW * L,), jnp.int32),
        pltpu.VMEM((_RPW * L,), jnp.float32),
        pltpu.VMEM((2, _LA, D), jnp.float32),
        pltpu.VMEM((2, _LB, D), jnp.float32),
        pltpu.VMEM((_RPW * D,), jnp.float32),
        pltpu.SemaphoreType.DMA,
        pltpu.SemaphoreType.DMA,
    ],
)(_bag_body)


_BL = 2048


def _cls_body(emb_ref, wt_ref, bt_ref, wc_ref, bc_ref, out_ref, h_ref):
    @pl.when(pl.program_id(0) == 0)
    def _():
        h = jnp.dot(emb_ref[...], wt_ref[...], preferred_element_type=jnp.float32)
        h_ref[...] = jnp.maximum(h + bt_ref[...], 0.0)

    out_ref[...] = lax.dot_general(
        h_ref[...], wc_ref[...],
        dimension_numbers=(((1,), (1,)), ((), ())),
        preferred_element_type=jnp.float32,
    ) + bc_ref[...]


_classify = pl.pallas_call(
    _cls_body,
    grid=(pl.cdiv(NUM_LABELS, _BL),),
    in_specs=[
        pl.BlockSpec((B, D), lambda j: (0, 0)),
        pl.BlockSpec((D, D), lambda j: (0, 0)),
        pl.BlockSpec((1, D), lambda j: (0, 0)),
        pl.BlockSpec((_BL, D), lambda j: (j, 0)),
        pl.BlockSpec((1, _BL), lambda j: (0, j)),
    ],
    out_specs=pl.BlockSpec((B, _BL), lambda j: (0, j)),
    out_shape=jax.ShapeDtypeStruct((B, NUM_LABELS), jnp.float32),
    scratch_shapes=[pltpu.VMEM((B, D), jnp.float32)],
)


def kernel(X, X_w, emb_table, W_t, b_t, W_c, b_c):
    embed = _bag(X.reshape(B * L), X_w.reshape(B * L), emb_table)
    return _classify(embed.reshape(B, D), W_t, b_t.reshape(1, D),
                     W_c, b_c.reshape(1, NUM_LABELS))
